# baseline (speedup 1.0000x reference)
import functools

import jax
import jax.numpy as jnp
from jax import lax
from jax.experimental import pallas as pl
from jax.experimental.pallas import tpu as pltpu
from jax.experimental.pallas import tpu_sc as plsc

N_NODES = 10000
N_PAD = 10240
D = 128
NC = 2
NS = 16
NW = NC * NS
LANES = 16
CHUNK = 128
ROWS_PER_TILE = N_PAD // NS
ZCOPY = 128
DEG_ROWS = N_PAD // D

_MESH = dict(core_axis_name="c", subcore_axis_name="s", num_cores=NC,
             num_subcores=NS)
_SC_PARAMS = pltpu.CompilerParams(needs_layout_passes=False)


def _worker_id():
    return lax.axis_index("c") * NS + lax.axis_index("s")


def _deg_body(epw, dst_hbm, w_hbm, deg_out, dstv, wv, degloc, redbuf, accbuf,
              slab):
    cid = lax.axis_index("c")
    sid = lax.axis_index("s")
    wid = _worker_id()

    def zrow(i, carry):
        for h in range(D // LANES):
            degloc[i, pl.ds(h * LANES, LANES)] = jnp.zeros((LANES,),
                                                           jnp.float32)
        return carry

    lax.fori_loop(0, DEG_ROWS, zrow, 0)

    base = wid * epw
    seven = jnp.full((LANES,), 7, jnp.int32)
    mask = jnp.full((LANES,), D - 1, jnp.int32)

    def body(k, carry):
        off = base + k * CHUNK
        pltpu.sync_copy(dst_hbm.at[pl.ds(off, CHUNK)], dstv)
        pltpu.sync_copy(w_hbm.at[pl.ds(off, CHUNK)], wv)
        for g in range(CHUNK // LANES):
            d16 = dstv[pl.ds(g * LANES, LANES)]
            w16 = wv[pl.ds(g * LANES, LANES)]
            row = lax.shift_right_logical(d16, seven)
            lane = jnp.bitwise_and(d16, mask)
            plsc.addupdate_scatter(degloc, [row, lane], w16)
        return carry

    lax.fori_loop(0, epw // CHUNK, body, 0)
    pltpu.sync_copy(degloc, slab.at[pl.ds(sid * DEG_ROWS, DEG_ROWS)])
    plsc.subcore_barrier()

    @pl.when(sid == 0)
    def _():
        pltpu.sync_copy(slab.at[pl.ds(0, DEG_ROWS)], accbuf)

        def radd(r, carry):
            for h in range(D // LANES):
                s = pl.ds(h * LANES, LANES)
                accbuf[r, s] = accbuf[r, s] + redbuf[r, s]
            return carry

        def tred(t, carry):
            pltpu.sync_copy(slab.at[pl.ds(t * DEG_ROWS, DEG_ROWS)], redbuf)
            lax.fori_loop(0, DEG_ROWS, radd, 0)
            return carry

        lax.fori_loop(1, NS, tred, 0)
        pltpu.sync_copy(accbuf, deg_out.at[cid])


MC = 64


def _msg_body(epw, y_hbm, src_hbm, dst_hbm, w_hbm, acc_out,
              srcv0, dstv0, wv0, rows0, srcv1, dstv1, wv1, rows1,
              acc, gsem0, gsem1, ssem0, ssem1):
    cid = lax.axis_index("c")
    sid = lax.axis_index("s")
    wid = _worker_id()
    nchunks = epw // MC

    def idx_load(off, srcv, dstv, wvb):
        pltpu.sync_copy(src_hbm.at[pl.ds(off, MC)], srcv)
        pltpu.sync_copy(dst_hbm.at[pl.ds(off, MC)], dstv)
        pltpu.sync_copy(w_hbm.at[pl.ds(off, MC)], wvb.at[pl.ds(LANES, MC)])

    def gather_start(srcv, rows, gsem):
        pltpu.async_copy(y_hbm.at[srcv], rows, gsem)

    def gather_wait(srcv, rows, gsem):
        pltpu.make_async_copy(y_hbm.at[srcv], rows, gsem).wait()

    def scale(rows, wvb):
        for e in range(MC):
            b = plsc.load_gather(
                wvb, [jnp.full((LANES,), LANES + e, jnp.int32)])
            for h in range(D // LANES):
                s = pl.ds(h * LANES, LANES)
                rows[e, s] = rows[e, s] * b

    def scatter_start(rows, dstv, ssem):
        pltpu.async_copy(rows, acc.at[dstv], ssem, add=True)

    def scatter_wait(rows, dstv, ssem):
        pltpu.make_async_copy(rows, acc.at[dstv], ssem).wait()

    def zrow(i, carry):
        for h in range(D // LANES):
            z = jnp.zeros((LANES,), jnp.float32)
            rows0[i, pl.ds(h * LANES, LANES)] = z
            rows1[i, pl.ds(h * LANES, LANES)] = z
        return carry

    lax.fori_loop(0, MC, zrow, 0)
    for t in range(ROWS_PER_TILE // MC):
        pltpu.sync_copy(rows0,
                        acc.at[pl.ds(sid * ROWS_PER_TILE + t * MC, MC)])
    plsc.subcore_barrier()

    base = wid * epw

    def zidx(i, carry):
        dstv1[pl.ds(i * LANES, LANES)] = jnp.zeros((LANES,), jnp.int32)
        return carry

    lax.fori_loop(0, MC // LANES, zidx, 0)
    scatter_start(rows1, dstv1, ssem1)
    idx_load(base, srcv0, dstv0, wv0)
    gather_start(srcv0, rows0, gsem0)

    def body(i, carry):
        a_off = base + (2 * i) * MC
        b_off = a_off + MC
        n_off = jnp.where(2 * i + 2 < nchunks, a_off + 2 * MC, base)
        gather_wait(srcv0, rows0, gsem0)
        scatter_wait(rows1, dstv1, ssem1)
        idx_load(b_off, srcv1, dstv1, wv1)
        gather_start(srcv1, rows1, gsem1)
        scale(rows0, wv0)
        scatter_start(rows0, dstv0, ssem0)
        gather_wait(srcv1, rows1, gsem1)
        scale(rows1, wv1)
        scatter_start(rows1, dstv1, ssem1)
        scatter_wait(rows0, dstv0, ssem0)
        idx_load(n_off, srcv0, dstv0, wv0)
        gather_start(srcv0, rows0, gsem0)
        return carry

    lax.fori_loop(0, nchunks // 2, body, 0)
    gather_wait(srcv0, rows0, gsem0)
    scatter_wait(rows1, dstv1, ssem1)
    plsc.subcore_barrier()
    sl = pl.ds(sid * ROWS_PER_TILE, ROWS_PER_TILE)
    pltpu.sync_copy(acc.at[sl], acc_out.at[cid, sl])


def _make_deg_kernel(epw):
    return pl.kernel(
        functools.partial(_deg_body, epw),
        out_type=jax.ShapeDtypeStruct((NC, DEG_ROWS, D), jnp.float32),
        mesh=plsc.VectorSubcoreMesh(**_MESH),
        scratch_types=[
            pltpu.VMEM((CHUNK,), jnp.int32),
            pltpu.VMEM((CHUNK,), jnp.float32),
            pltpu.VMEM((DEG_ROWS, D), jnp.float32),
            pltpu.VMEM((DEG_ROWS, D), jnp.float32),
            pltpu.VMEM((DEG_ROWS, D), jnp.float32),
            pltpu.VMEM_SHARED((NS * DEG_ROWS, D), jnp.float32),
        ],
        compiler_params=_SC_PARAMS,
    )


def _make_msg_kernel(epw):
    return pl.kernel(
        functools.partial(_msg_body, epw),
        out_type=jax.ShapeDtypeStruct((NC, N_PAD, D), jnp.float32),
        mesh=plsc.VectorSubcoreMesh(**_MESH),
        scratch_types=[
            pltpu.VMEM((MC,), jnp.int32),
            pltpu.VMEM((MC,), jnp.int32),
            pltpu.VMEM((MC + LANES,), jnp.float32),
            pltpu.VMEM((MC, D), jnp.float32),
            pltpu.VMEM((MC,), jnp.int32),
            pltpu.VMEM((MC,), jnp.int32),
            pltpu.VMEM((MC + LANES,), jnp.float32),
            pltpu.VMEM((MC, D), jnp.float32),
            pltpu.VMEM_SHARED((N_PAD, D), jnp.float32),
            pltpu.SemaphoreType.DMA,
            pltpu.SemaphoreType.DMA,
            pltpu.SemaphoreType.DMA,
            pltpu.SemaphoreType.DMA,
        ],
        compiler_params=_SC_PARAMS,
    )


ROW_BLK = 2000


def _prep_body(x_ref, w_ref, degp_ref, xw_ref, y_ref, dinv_ref, dinv2_ref):
    deg = degp_ref[0] + degp_ref[1] + 1.0
    dinv = jnp.where(deg > 0, lax.rsqrt(jnp.maximum(deg, 1e-12)),
                     jnp.zeros_like(deg))
    xw = jnp.dot(x_ref[...], w_ref[...], preferred_element_type=jnp.float32)
    xw_ref[...] = xw
    y_ref[...] = xw * dinv
    dinv_ref[...] = dinv
    dinv2_ref[...] = dinv * dinv


def _prep_call(x, w1, degp):
    grid = (N_NODES // ROW_BLK,)
    return pl.pallas_call(
        _prep_body,
        grid=grid,
        in_specs=[
            pl.BlockSpec((ROW_BLK, D), lambda r: (r, 0)),
            pl.BlockSpec((D, D), lambda r: (0, 0)),
            pl.BlockSpec((NC, ROW_BLK, 1), lambda r: (0, r, 0)),
        ],
        out_specs=[
            pl.BlockSpec((ROW_BLK, D), lambda r: (r, 0)),
            pl.BlockSpec((ROW_BLK, D), lambda r: (r, 0)),
            pl.BlockSpec((ROW_BLK, 1), lambda r: (r, 0)),
            pl.BlockSpec((ROW_BLK, 1), lambda r: (r, 0)),
        ],
        out_shape=[
            jax.ShapeDtypeStruct((N_NODES, D), jnp.float32),
            jax.ShapeDtypeStruct((N_NODES, D), jnp.float32),
            jax.ShapeDtypeStruct((N_NODES, 1), jnp.float32),
            jax.ShapeDtypeStruct((N_NODES, 1), jnp.float32),
        ],
    )(x, w1, degp)


def _comb_mm_body(accp_ref, xw_ref, dinv_ref, dinv2_ref, b_ref, w2_ref,
                  x1_ref, xw2_ref, y2_ref):
    s = accp_ref[0] + accp_ref[1]
    x1 = jnp.maximum(dinv_ref[...] * s + dinv2_ref[...] * xw_ref[...]
                     + b_ref[...], 0.0)
    x1_ref[...] = x1
    xw2 = jnp.dot(x1, w2_ref[...], preferred_element_type=jnp.float32)
    xw2_ref[...] = xw2
    y2_ref[...] = xw2 * dinv_ref[...]


def _comb_mm_call(accp, xw1, dinv, dinv2, b1, w2):
    grid = (N_NODES // ROW_BLK,)
    return pl.pallas_call(
        _comb_mm_body,
        grid=grid,
        in_specs=[
            pl.BlockSpec((NC, ROW_BLK, D), lambda r: (0, r, 0)),
            pl.BlockSpec((ROW_BLK, D), lambda r: (r, 0)),
            pl.BlockSpec((ROW_BLK, 1), lambda r: (r, 0)),
            pl.BlockSpec((ROW_BLK, 1), lambda r: (r, 0)),
            pl.BlockSpec((1, D), lambda r: (0, 0)),
            pl.BlockSpec((D, D), lambda r: (0, 0)),
        ],
        out_specs=[
            pl.BlockSpec((ROW_BLK, D), lambda r: (r, 0)),
            pl.BlockSpec((ROW_BLK, D), lambda r: (r, 0)),
            pl.BlockSpec((ROW_BLK, D), lambda r: (r, 0)),
        ],
        out_shape=[
            jax.ShapeDtypeStruct((N_NODES, D), jnp.float32),
            jax.ShapeDtypeStruct((N_NODES, D), jnp.float32),
            jax.ShapeDtypeStruct((N_NODES, D), jnp.float32),
        ],
    )(accp, xw1, dinv, dinv2, b1, w2)


def _comb_body(accp_ref, xw_ref, dinv_ref, dinv2_ref, b_ref, x2_ref):
    s = accp_ref[0] + accp_ref[1]
    x2_ref[...] = jnp.maximum(dinv_ref[...] * s + dinv2_ref[...] * xw_ref[...]
                              + b_ref[...], 0.0)


def _comb_call(accp, xw2, dinv, dinv2, b2):
    grid = (N_NODES // ROW_BLK,)
    return pl.pallas_call(
        _comb_body,
        grid=grid,
        in_specs=[
            pl.BlockSpec((NC, ROW_BLK, D), lambda r: (0, r, 0)),
            pl.BlockSpec((ROW_BLK, D), lambda r: (r, 0)),
            pl.BlockSpec((ROW_BLK, 1), lambda r: (r, 0)),
            pl.BlockSpec((ROW_BLK, 1), lambda r: (r, 0)),
            pl.BlockSpec((1, D), lambda r: (0, 0)),
        ],
        out_specs=pl.BlockSpec((ROW_BLK, D), lambda r: (r, 0)),
        out_shape=jax.ShapeDtypeStruct((N_NODES, D), jnp.float32),
    )(accp, xw2, dinv, dinv2, b2)


def kernel(x, edge_index, edge_weight, W1, b1, W2, b2):
    n_edges = edge_index.shape[1]
    quantum = max(CHUNK, 2 * MC)
    epw = ((n_edges + NW * quantum - 1) // (NW * quantum)) * quantum
    pad = epw * NW - n_edges

    src = edge_index[0].astype(jnp.int32)
    dst = edge_index[1].astype(jnp.int32)
    w = edge_weight.astype(jnp.float32)
    if pad:
        zi = jnp.zeros((pad,), jnp.int32)
        src = jnp.concatenate([src, zi])
        dst = jnp.concatenate([dst, zi])
        w = jnp.concatenate([w, jnp.zeros((pad,), jnp.float32)])

    degp = _make_deg_kernel(epw)(dst, w).reshape(NC, N_PAD, 1)
    b1r = b1.reshape(1, D)
    b2r = b2.reshape(1, D)

    xw1, y1, dinv, dinv2 = _prep_call(x, W1, degp)
    accp1 = _make_msg_kernel(epw)(y1, src, dst, w)
    x1, xw2, y2 = _comb_mm_call(accp1, xw1, dinv, dinv2, b1r, W2)
    accp2 = _make_msg_kernel(epw)(y2, src, dst, w)
    x2 = _comb_call(accp2, xw2, dinv, dinv2, b2r)

    return jnp.concatenate([x1[:, :, None], x2[:, :, None]], axis=2)


# speedup vs baseline: 1.0119x; 1.0119x over previous
import functools

import jax
import jax.numpy as jnp
from jax import lax
from jax.experimental import pallas as pl
from jax.experimental.pallas import tpu as pltpu
from jax.experimental.pallas import tpu_sc as plsc

N_NODES = 10000
N_PAD = 10240
D = 128
NC = 2
NS = 16
NW = NC * NS
LANES = 16
CHUNK = 128
ROWS_PER_TILE = N_PAD // NS
ZCOPY = 128
DEG_ROWS = N_PAD // D

_MESH = dict(core_axis_name="c", subcore_axis_name="s", num_cores=NC,
             num_subcores=NS)
_SC_PARAMS = pltpu.CompilerParams(needs_layout_passes=False)


def _worker_id():
    return lax.axis_index("c") * NS + lax.axis_index("s")


def _deg_body(epw, dst_hbm, w_hbm, deg_out, dstv, wv, degloc, redbuf, accbuf,
              slab):
    cid = lax.axis_index("c")
    sid = lax.axis_index("s")
    wid = _worker_id()

    def zrow(i, carry):
        for h in range(D // LANES):
            degloc[i, pl.ds(h * LANES, LANES)] = jnp.zeros((LANES,),
                                                           jnp.float32)
        return carry

    lax.fori_loop(0, DEG_ROWS, zrow, 0)

    base = wid * epw
    seven = jnp.full((LANES,), 7, jnp.int32)
    mask = jnp.full((LANES,), D - 1, jnp.int32)

    def body(k, carry):
        off = base + k * CHUNK
        pltpu.sync_copy(dst_hbm.at[pl.ds(off, CHUNK)], dstv)
        pltpu.sync_copy(w_hbm.at[pl.ds(off, CHUNK)], wv)
        for g in range(CHUNK // LANES):
            d16 = dstv[pl.ds(g * LANES, LANES)]
            w16 = wv[pl.ds(g * LANES, LANES)]
            row = lax.shift_right_logical(d16, seven)
            lane = jnp.bitwise_and(d16, mask)
            plsc.addupdate_scatter(degloc, [row, lane], w16)
        return carry

    lax.fori_loop(0, epw // CHUNK, body, 0)
    pltpu.sync_copy(degloc, slab.at[pl.ds(sid * DEG_ROWS, DEG_ROWS)])
    plsc.subcore_barrier()

    @pl.when(sid == 0)
    def _():
        pltpu.sync_copy(slab.at[pl.ds(0, DEG_ROWS)], accbuf)

        def radd(r, carry):
            for h in range(D // LANES):
                s = pl.ds(h * LANES, LANES)
                accbuf[r, s] = accbuf[r, s] + redbuf[r, s]
            return carry

        def tred(t, carry):
            pltpu.sync_copy(slab.at[pl.ds(t * DEG_ROWS, DEG_ROWS)], redbuf)
            lax.fori_loop(0, DEG_ROWS, radd, 0)
            return carry

        lax.fori_loop(1, NS, tred, 0)
        pltpu.sync_copy(accbuf, deg_out.at[cid])


def _msg_body(epw, y_hbm, src_hbm, dst_hbm, w_hbm, acc_out,
              srcv, dstv, wv, rows, acc, sem):
    cid = lax.axis_index("c")
    sid = lax.axis_index("s")
    wid = _worker_id()

    def zrow(i, carry):
        for h in range(D // LANES):
            rows[i, pl.ds(h * LANES, LANES)] = jnp.zeros((LANES,), jnp.float32)
        return carry

    lax.fori_loop(0, CHUNK, zrow, 0)
    for t in range(ROWS_PER_TILE // ZCOPY):
        pltpu.sync_copy(rows.at[pl.ds(0, ZCOPY)],
                        acc.at[pl.ds(sid * ROWS_PER_TILE + t * ZCOPY, ZCOPY)])
    plsc.subcore_barrier()

    base = wid * epw

    def body(k, carry):
        off = base + k * CHUNK
        pltpu.sync_copy(src_hbm.at[pl.ds(off, CHUNK)], srcv)
        pltpu.sync_copy(dst_hbm.at[pl.ds(off, CHUNK)], dstv)
        pltpu.sync_copy(w_hbm.at[pl.ds(off, CHUNK)],
                        wv.at[pl.ds(LANES, CHUNK)])
        pltpu.async_copy(y_hbm.at[srcv], rows, sem).wait()
        for e in range(CHUNK):
            b = plsc.load_gather(
                wv, [jnp.full((LANES,), LANES + e, jnp.int32)])
            for h in range(D // LANES):
                s = pl.ds(h * LANES, LANES)
                rows[e, s] = rows[e, s] * b
        pltpu.sync_copy(rows, acc.at[dstv], add=True)
        return carry

    lax.fori_loop(0, epw // CHUNK, body, 0)
    plsc.subcore_barrier()
    sl = pl.ds(sid * ROWS_PER_TILE, ROWS_PER_TILE)
    pltpu.sync_copy(acc.at[sl], acc_out.at[cid, sl])


def _make_deg_kernel(epw):
    return pl.kernel(
        functools.partial(_deg_body, epw),
        out_type=jax.ShapeDtypeStruct((NC, DEG_ROWS, D), jnp.float32),
        mesh=plsc.VectorSubcoreMesh(**_MESH),
        scratch_types=[
            pltpu.VMEM((CHUNK,), jnp.int32),
            pltpu.VMEM((CHUNK,), jnp.float32),
            pltpu.VMEM((DEG_ROWS, D), jnp.float32),
            pltpu.VMEM((DEG_ROWS, D), jnp.float32),
            pltpu.VMEM((DEG_ROWS, D), jnp.float32),
            pltpu.VMEM_SHARED((NS * DEG_ROWS, D), jnp.float32),
        ],
        compiler_params=_SC_PARAMS,
    )


def _make_msg_kernel(epw):
    return pl.kernel(
        functools.partial(_msg_body, epw),
        out_type=jax.ShapeDtypeStruct((NC, N_PAD, D), jnp.float32),
        mesh=plsc.VectorSubcoreMesh(**_MESH),
        scratch_types=[
            pltpu.VMEM((CHUNK,), jnp.int32),
            pltpu.VMEM((CHUNK,), jnp.int32),
            pltpu.VMEM((CHUNK + LANES,), jnp.float32),
            pltpu.VMEM((CHUNK, D), jnp.float32),
            pltpu.VMEM_SHARED((N_PAD, D), jnp.float32),
            pltpu.SemaphoreType.DMA,
        ],
        compiler_params=_SC_PARAMS,
    )


ROW_BLK = 2000


def _prep_body(x_ref, w_ref, degp_ref, xw_ref, y_ref, dinv_ref, dinv2_ref):
    deg = degp_ref[0] + degp_ref[1] + 1.0
    dinv = jnp.where(deg > 0, lax.rsqrt(jnp.maximum(deg, 1e-12)),
                     jnp.zeros_like(deg))
    xw = jnp.dot(x_ref[...], w_ref[...], preferred_element_type=jnp.float32)
    xw_ref[...] = xw
    y_ref[...] = xw * dinv
    dinv_ref[...] = dinv
    dinv2_ref[...] = dinv * dinv


def _prep_call(x, w1, degp):
    grid = (N_NODES // ROW_BLK,)
    return pl.pallas_call(
        _prep_body,
        grid=grid,
        in_specs=[
            pl.BlockSpec((ROW_BLK, D), lambda r: (r, 0)),
            pl.BlockSpec((D, D), lambda r: (0, 0)),
            pl.BlockSpec((NC, ROW_BLK, 1), lambda r: (0, r, 0)),
        ],
        out_specs=[
            pl.BlockSpec((ROW_BLK, D), lambda r: (r, 0)),
            pl.BlockSpec((ROW_BLK, D), lambda r: (r, 0)),
            pl.BlockSpec((ROW_BLK, 1), lambda r: (r, 0)),
            pl.BlockSpec((ROW_BLK, 1), lambda r: (r, 0)),
        ],
        out_shape=[
            jax.ShapeDtypeStruct((N_NODES, D), jnp.float32),
            jax.ShapeDtypeStruct((N_NODES, D), jnp.float32),
            jax.ShapeDtypeStruct((N_NODES, 1), jnp.float32),
            jax.ShapeDtypeStruct((N_NODES, 1), jnp.float32),
        ],
    )(x, w1, degp)


def _comb_mm_body(accp_ref, xw_ref, dinv_ref, dinv2_ref, b_ref, w2_ref,
                  x1_ref, xw2_ref, y2_ref):
    s = accp_ref[0] + accp_ref[1]
    x1 = jnp.maximum(dinv_ref[...] * s + dinv2_ref[...] * xw_ref[...]
                     + b_ref[...], 0.0)
    x1_ref[...] = x1
    xw2 = jnp.dot(x1, w2_ref[...], preferred_element_type=jnp.float32)
    xw2_ref[...] = xw2
    y2_ref[...] = xw2 * dinv_ref[...]


def _comb_mm_call(accp, xw1, dinv, dinv2, b1, w2):
    grid = (N_NODES // ROW_BLK,)
    return pl.pallas_call(
        _comb_mm_body,
        grid=grid,
        in_specs=[
            pl.BlockSpec((NC, ROW_BLK, D), lambda r: (0, r, 0)),
            pl.BlockSpec((ROW_BLK, D), lambda r: (r, 0)),
            pl.BlockSpec((ROW_BLK, 1), lambda r: (r, 0)),
            pl.BlockSpec((ROW_BLK, 1), lambda r: (r, 0)),
            pl.BlockSpec((1, D), lambda r: (0, 0)),
            pl.BlockSpec((D, D), lambda r: (0, 0)),
        ],
        out_specs=[
            pl.BlockSpec((ROW_BLK, D), lambda r: (r, 0)),
            pl.BlockSpec((ROW_BLK, D), lambda r: (r, 0)),
            pl.BlockSpec((ROW_BLK, D), lambda r: (r, 0)),
        ],
        out_shape=[
            jax.ShapeDtypeStruct((N_NODES, D), jnp.float32),
            jax.ShapeDtypeStruct((N_NODES, D), jnp.float32),
            jax.ShapeDtypeStruct((N_NODES, D), jnp.float32),
        ],
    )(accp, xw1, dinv, dinv2, b1, w2)


def _comb_body(accp_ref, xw_ref, dinv_ref, dinv2_ref, b_ref, x2_ref):
    s = accp_ref[0] + accp_ref[1]
    x2_ref[...] = jnp.maximum(dinv_ref[...] * s + dinv2_ref[...] * xw_ref[...]
                              + b_ref[...], 0.0)


def _comb_call(accp, xw2, dinv, dinv2, b2):
    grid = (N_NODES // ROW_BLK,)
    return pl.pallas_call(
        _comb_body,
        grid=grid,
        in_specs=[
            pl.BlockSpec((NC, ROW_BLK, D), lambda r: (0, r, 0)),
            pl.BlockSpec((ROW_BLK, D), lambda r: (r, 0)),
            pl.BlockSpec((ROW_BLK, 1), lambda r: (r, 0)),
            pl.BlockSpec((ROW_BLK, 1), lambda r: (r, 0)),
            pl.BlockSpec((1, D), lambda r: (0, 0)),
        ],
        out_specs=pl.BlockSpec((ROW_BLK, D), lambda r: (r, 0)),
        out_shape=jax.ShapeDtypeStruct((N_NODES, D), jnp.float32),
    )(accp, xw2, dinv, dinv2, b2)


def kernel(x, edge_index, edge_weight, W1, b1, W2, b2):
    n_edges = edge_index.shape[1]
    epw = ((n_edges + NW * CHUNK - 1) // (NW * CHUNK)) * CHUNK
    pad = epw * NW - n_edges

    src = edge_index[0].astype(jnp.int32)
    dst = edge_index[1].astype(jnp.int32)
    w = edge_weight.astype(jnp.float32)
    if pad:
        zi = jnp.zeros((pad,), jnp.int32)
        src = jnp.concatenate([src, zi])
        dst = jnp.concatenate([dst, zi])
        w = jnp.concatenate([w, jnp.zeros((pad,), jnp.float32)])

    degp = _make_deg_kernel(epw)(dst, w).reshape(NC, N_PAD, 1)
    b1r = b1.reshape(1, D)
    b2r = b2.reshape(1, D)

    xw1, y1, dinv, dinv2 = _prep_call(x, W1, degp)
    accp1 = _make_msg_kernel(epw)(y1, src, dst, w)
    x1, xw2, y2 = _comb_mm_call(accp1, xw1, dinv, dinv2, b1r, W2)
    accp2 = _make_msg_kernel(epw)(y2, src, dst, w)
    x2 = _comb_call(accp2, xw2, dinv, dinv2, b2r)

    return jnp.concatenate([x1[:, :, None], x2[:, :, None]], axis=2)
